# speedup vs baseline: 6.9757x; 6.9757x over previous
import functools

import jax
import jax.numpy as jnp
from jax.experimental import pallas as pl
from jax.experimental.pallas import tpu as pltpu

BN = 1024
BM = 1024
NEG = -1e30


def _flash_kernel(n_real, n_col_blocks,
                  x_row_ref, x_col_ref, a_ref, w_ref, as_ref, an_ref, b_ref,
                  out_ref, acc_ref, m_ref, l_ref, s_ref, ws_ref, wt_ref):
    i = pl.program_id(0)
    j = pl.program_id(1)

    @pl.when(jnp.logical_and(i == 0, j == 0))
    def _init_weights():
        ws_ref[...] = jnp.dot(w_ref[...], as_ref[...],
                              preferred_element_type=jnp.float32)
        wt_ref[...] = jnp.dot(w_ref[...], an_ref[...],
                              preferred_element_type=jnp.float32)

    @pl.when(j == 0)
    def _init_row_block():
        s_ref[...] = jnp.dot(x_row_ref[...], ws_ref[...],
                             preferred_element_type=jnp.float32)
        m_ref[...] = jnp.full_like(m_ref, NEG)
        l_ref[...] = jnp.zeros_like(l_ref)
        acc_ref[...] = jnp.zeros_like(acc_ref)

    x_col = x_col_ref[...]
    t_col = jnp.dot(x_col, wt_ref[...],
                    preferred_element_type=jnp.float32)
    t_row = t_col.reshape(1, BM)

    z = s_ref[...] + t_row
    logit = jnp.where(z >= 0.0, z, 0.2 * z)

    row_ids = i * BN + jax.lax.broadcasted_iota(jnp.int32, (BN, BM), 0)
    col_ids = j * BM + jax.lax.broadcasted_iota(jnp.int32, (BN, BM), 1)
    valid = jnp.logical_and(
        col_ids < n_real,
        jnp.logical_or(a_ref[...] != 0.0, row_ids == col_ids))

    logit = jnp.where(valid, logit, NEG)
    m_old = m_ref[...]
    m_new = jnp.maximum(m_old, jnp.max(logit, axis=1, keepdims=True))
    p = jnp.where(valid, jnp.exp(logit - m_new), 0.0)
    scale = jnp.exp(m_old - m_new)
    l_ref[...] = l_ref[...] * scale + jnp.sum(p, axis=1, keepdims=True)
    acc_ref[...] = acc_ref[...] * scale + jnp.dot(
        p, x_col, preferred_element_type=jnp.float32)
    m_ref[...] = m_new

    @pl.when(j == n_col_blocks - 1)
    def _finalize():
        out_ref[...] = jnp.dot(acc_ref[...] / l_ref[...], w_ref[...],
                               preferred_element_type=jnp.float32) + b_ref[...]


@jax.jit
def kernel(x, a, kernel, attn_kernel_self, attn_kernel_neighs, bias):
    n, i_dim = x.shape
    o_dim = kernel.shape[2]
    w = kernel.reshape(i_dim, o_dim)
    a_s = attn_kernel_self.reshape(o_dim, 1)
    a_n = attn_kernel_neighs.reshape(o_dim, 1)
    b = bias.reshape(1, o_dim)

    n_row_blocks = pl.cdiv(n, BN)
    n_col_blocks = pl.cdiv(n, BM)
    n_pad = max(n_row_blocks * BN, n_col_blocks * BM)
    x_p = jnp.pad(x, ((0, n_pad - n), (0, 0)))

    grid = (n_row_blocks, n_col_blocks)
    out = pl.pallas_call(
        functools.partial(_flash_kernel, n, n_col_blocks),
        grid=grid,
        in_specs=[
            pl.BlockSpec((BN, i_dim), lambda i, j: (i, 0)),
            pl.BlockSpec((BM, i_dim), lambda i, j: (j, 0)),
            pl.BlockSpec((BN, BM), lambda i, j: (i, j)),
            pl.BlockSpec((i_dim, o_dim), lambda i, j: (0, 0)),
            pl.BlockSpec((o_dim, 1), lambda i, j: (0, 0)),
            pl.BlockSpec((o_dim, 1), lambda i, j: (0, 0)),
            pl.BlockSpec((1, o_dim), lambda i, j: (0, 0)),
        ],
        out_specs=pl.BlockSpec((BN, o_dim), lambda i, j: (i, 0)),
        out_shape=jax.ShapeDtypeStruct((n, o_dim), jnp.float32),
        scratch_shapes=[
            pltpu.VMEM((BN, o_dim), jnp.float32),
            pltpu.VMEM((BN, 1), jnp.float32),
            pltpu.VMEM((BN, 1), jnp.float32),
            pltpu.VMEM((BN, 1), jnp.float32),
            pltpu.VMEM((i_dim, 1), jnp.float32),
            pltpu.VMEM((i_dim, 1), jnp.float32),
        ],
        compiler_params=pltpu.CompilerParams(
            dimension_semantics=("arbitrary", "arbitrary")),
    )(x_p, x_p, a, w, a_s, a_n, b)
    return out
